# baseline (speedup 1.0000x reference)
import functools
import math

import jax
import jax.numpy as jnp
from jax import lax
from jax.experimental import pallas as pl
from jax.experimental.pallas import tpu as pltpu
from jax.experimental.pallas import tpu_sc as plsc

VOCAB = 1000
D_MODEL = 32
SCALE = math.sqrt(float(D_MODEL))

NUM_CORES = 2
NUM_SUBCORES = 16
NW = NUM_CORES * NUM_SUBCORES

BATCH = 16384
SEQ = 200
W_TILE = BATCH // NW

R_STAGE = 40
N_STAGE = SEQ // R_STAGE
LANES = 16
CW = 256
N_GROUPS = CW // LANES
K_CHUNKS = W_TILE // CW


def _scale_body(t_ref, o_ref):
    o_ref[...] = t_ref[...].T * SCALE


def _scale_table(table):
    return pl.pallas_call(
        _scale_body,
        out_shape=jax.ShapeDtypeStruct((D_MODEL, VOCAB), table.dtype),
    )(table)


@functools.partial(
    pl.kernel,
    mesh=plsc.VectorSubcoreMesh(core_axis_name="c", subcore_axis_name="s"),
    out_type=jax.ShapeDtypeStruct((SEQ, D_MODEL, BATCH), jnp.float32),
    scratch_types=[
        pltpu.VMEM((VOCAB * D_MODEL,), jnp.float32),
        pltpu.VMEM((R_STAGE, W_TILE), jnp.int32),
        pltpu.VMEM((R_STAGE, W_TILE), jnp.int32),
        pltpu.VMEM((D_MODEL, CW), jnp.float32),
        pltpu.VMEM((D_MODEL, CW), jnp.float32),
        pltpu.SemaphoreType.DMA,
        pltpu.SemaphoreType.DMA,
        pltpu.SemaphoreType.DMA,
        pltpu.SemaphoreType.DMA,
    ],
    compiler_params=pltpu.CompilerParams(use_tc_tiling_on_sc=True,
                                         needs_layout_passes=False),
)
def _gather_t(table_hbm, xt_hbm, out_hbm,
              table_v, xs0, xs1, oc0, oc1,
              xsem0, xsem1, osem0, osem1):
    xs = (xs0, xs1)
    oc = (oc0, oc1)
    xsem = (xsem0, xsem1)
    osem = (osem0, osem1)

    wid = lax.axis_index("s") * NUM_CORES + lax.axis_index("c")
    col0 = wid * W_TILE

    pltpu.sync_copy(table_hbm, table_v)

    def issue_stage(s, b):
        pltpu.async_copy(
            xt_hbm.at[pl.ds(s * R_STAGE, R_STAGE), pl.ds(col0, W_TILE)],
            xs[b], xsem[b])

    def wait_stage(b):
        pltpu.make_async_copy(
            xt_hbm.at[pl.ds(0, R_STAGE), pl.ds(0, W_TILE)],
            xs[b], xsem[b]).wait()

    def wait_out(slot):
        pltpu.make_async_copy(
            oc[slot], out_hbm.at[0, :, pl.ds(0, CW)], osem[slot]).wait()

    def compute_chunk(r, k, b, slot):
        @plsc.parallel_loop(0, N_GROUPS)
        def group(g):
            off = pl.multiple_of(k * CW + g * LANES, LANES)
            idxv = xs[b][r, pl.ds(off, LANES)]
            vals = [plsc.load_gather(table_v, [idxv + c * VOCAB])
                    for c in range(D_MODEL)]
            for c in range(D_MODEL):
                oc[slot][c, pl.ds(g * LANES, LANES)] = vals[c]

    def issue_chunk(i1, k, slot):
        pltpu.async_copy(
            oc[slot],
            out_hbm.at[i1, :, pl.ds(col0 + k * CW, CW)],
            osem[slot])

    issue_stage(0, 0)
    issue_stage(1, 1)

    for s in range(N_STAGE):
        b = s % 2
        wait_stage(b)

        def row_body(r, _, s=s, b=b):
            i1 = s * R_STAGE + r
            for k in range(K_CHUNKS):
                slot = k % 2
                if s == 0 and k < 2:
                    @pl.when(r > 0)
                    def _():
                        wait_out(slot)
                else:
                    wait_out(slot)
                compute_chunk(r, k, b, slot)
                issue_chunk(i1, k, slot)
            return 0

        lax.fori_loop(0, R_STAGE, row_body, 0, unroll=False)
        if s + 2 < N_STAGE:
            issue_stage(s + 2, b)

    wait_out(0)
    wait_out(1)


def kernel(x, table):
    xt = x.T.astype(jnp.int32)
    flat = _scale_table(table).reshape(-1)
    out_t = _gather_t(flat, xt)
    return out_t.transpose(2, 0, 1)
